# baseline (speedup 1.0000x reference)
import functools

import jax
import jax.numpy as jnp
from jax import lax
from jax.experimental import pallas as pl
from jax.experimental.pallas import tpu as pltpu
from jax.experimental.pallas import tpu_sc as plsc

D = 1024
D2 = D // 2
NW = 32
CHUNK = 32
L = 16


def _normalize_pack_body(t_ref, o_ref):
    x = t_ref[...]
    s = jnp.sum(x * x, axis=1, keepdims=True)
    inv = 1.0 / jnp.maximum(jnp.sqrt(s), 1e-12)
    xn = x * inv
    lo = jax.lax.bitcast_convert_type(
        xn[:, :D2].astype(jnp.bfloat16), jnp.uint16).astype(jnp.uint32)
    hi = jax.lax.bitcast_convert_type(
        xn[:, D2:].astype(jnp.bfloat16), jnp.uint16).astype(jnp.uint32)
    o_ref[...] = jax.lax.bitcast_convert_type(lo | (hi << 16), jnp.int32)


def _normalize_pack_table(table):
    rows, d = table.shape
    blk = 512
    return pl.pallas_call(
        _normalize_pack_body,
        grid=(rows // blk,),
        in_specs=[pl.BlockSpec((blk, d), lambda i: (i, 0))],
        out_specs=pl.BlockSpec((blk, d // 2), lambda i: (i, 0)),
        out_shape=jax.ShapeDtypeStruct((rows, d // 2), jnp.int32),
    )(table)


def _make_sc_gather(n_rows_total):
    n_per_w = n_rows_total // NW
    n_chunks = n_per_w // CHUNK
    assert n_chunks % 2 == 0 and n_chunks >= 6
    mesh = plsc.VectorSubcoreMesh(core_axis_name="c", subcore_axis_name="s")

    @functools.partial(
        pl.kernel,
        mesh=mesh,
        out_type=jax.ShapeDtypeStruct((n_rows_total, D), jnp.int32),
        scratch_types=[
            pltpu.VMEM((n_chunks, CHUNK), jnp.int32),
            pltpu.VMEM((2, CHUNK, D2), jnp.int32),
            pltpu.VMEM((2, CHUNK, D), jnp.int32),
            pltpu.SemaphoreType.DMA,
            pltpu.SemaphoreType.DMA,
            pltpu.SemaphoreType.DMA,
            pltpu.SemaphoreType.DMA,
        ],
    )
    def gather_kernel(table_hbm, idx_hbm, out_hbm, idx_v, gbuf, fbuf,
                      g0, g1, s0, s1):
        gs = (g0, g1)
        ss = (s0, s1)
        wid = lax.axis_index("s") * 2 + lax.axis_index("c")
        pltpu.sync_copy(idx_hbm.at[wid], idx_v)
        base = wid * n_per_w
        mask_hi = jnp.int32(-65536)

        def start_gather(jf, b):
            pltpu.async_copy(table_hbm.at[idx_v.at[jf]], gbuf.at[b], gs[b])

        def wait_gather(j, b):
            pltpu.make_async_copy(table_hbm.at[idx_v.at[j]], gbuf.at[b],
                                  gs[b]).wait()

        def start_scatter(j, b):
            pltpu.async_copy(fbuf.at[b],
                             out_hbm.at[pl.ds(base + j * CHUNK, CHUNK)], ss[b])

        def wait_scatter(j, b):
            pltpu.make_async_copy(fbuf.at[b],
                                  out_hbm.at[pl.ds(base + j * CHUNK, CHUNK)],
                                  ss[b]).wait()

        def expand(b):
            g = gbuf.at[b]
            f = fbuf.at[b]

            def row(r, carry):
                for sl in range(D2 // L):
                    w = g[r, pl.ds(sl * L, L)]
                    f[r, pl.ds(sl * L, L)] = w << 16
                    f[r, pl.ds(D2 + sl * L, L)] = w & mask_hi
                return carry

            lax.fori_loop(0, CHUNK, row, 0)

        def step(j, b, first, last):
            wait_gather(j, b)
            if not first:
                wait_scatter(j - 2, b)
            expand(b)
            start_scatter(j, b)
            if not last:
                start_gather(j + 2, b)

        start_gather(0, 0)
        start_gather(1, 1)
        step(0, 0, True, False)
        step(1, 1, True, False)

        def body(kk, carry):
            step(kk * 2, 0, False, False)
            step(kk * 2 + 1, 1, False, False)
            return carry

        lax.fori_loop(1, n_chunks // 2 - 1, body, 0)

        step(n_chunks - 2, 0, False, True)
        step(n_chunks - 1, 1, False, True)
        wait_scatter(n_chunks - 2, 0)
        wait_scatter(n_chunks - 1, 1)

    return gather_kernel


def kernel(positions, positional_embeddings):
    b = positions.size
    n_per_w = b // NW
    n_chunks = n_per_w // CHUNK
    packed = _normalize_pack_table(positional_embeddings)
    idx = positions.reshape(NW, n_chunks, CHUNK).astype(jnp.int32)
    out = _make_sc_gather(b)(packed, idx)
    out = jax.lax.bitcast_convert_type(out, jnp.float32)
    return out.reshape(positions.shape + (D,))


# speedup vs baseline: 1.3816x; 1.3816x over previous
import functools

import jax
import jax.numpy as jnp
from jax import lax
from jax.experimental import pallas as pl
from jax.experimental.pallas import tpu as pltpu
from jax.experimental.pallas import tpu_sc as plsc

D = 1024
D2 = D // 2
NW = 32
CHUNK = 32
L = 16


def _normalize_pack_body(t_ref, o_ref):
    x = t_ref[...]
    s = jnp.sum(x * x, axis=1, keepdims=True)
    inv = 1.0 / jnp.maximum(jnp.sqrt(s), 1e-12)
    xn = x * inv
    lo = jax.lax.bitcast_convert_type(
        xn[:, :D2].astype(jnp.bfloat16), jnp.uint16).astype(jnp.uint32)
    hi = jax.lax.bitcast_convert_type(
        xn[:, D2:].astype(jnp.bfloat16), jnp.uint16).astype(jnp.uint32)
    o_ref[...] = jax.lax.bitcast_convert_type(lo | (hi << 16), jnp.int32)


def _normalize_pack_table(table):
    rows, d = table.shape
    blk = 512
    return pl.pallas_call(
        _normalize_pack_body,
        grid=(rows // blk,),
        in_specs=[pl.BlockSpec((blk, d), lambda i: (i, 0))],
        out_specs=pl.BlockSpec((blk, d // 2), lambda i: (i, 0)),
        out_shape=jax.ShapeDtypeStruct((rows, d // 2), jnp.int32),
    )(table)


def _make_sc_gather(n_rows_total):
    n_per_w = n_rows_total // NW
    n_chunks = n_per_w // CHUNK
    assert n_chunks % 2 == 0 and n_chunks >= 6
    mesh = plsc.VectorSubcoreMesh(core_axis_name="c", subcore_axis_name="s")

    @functools.partial(
        pl.kernel,
        mesh=mesh,
        out_type=jax.ShapeDtypeStruct((n_rows_total, D), jnp.int32),
        scratch_types=[
            pltpu.VMEM((n_chunks, CHUNK), jnp.int32),
            pltpu.VMEM((2, CHUNK, D2), jnp.int32),
            pltpu.VMEM((2, CHUNK, D), jnp.int32),
            pltpu.SemaphoreType.DMA,
            pltpu.SemaphoreType.DMA,
            pltpu.SemaphoreType.DMA,
            pltpu.SemaphoreType.DMA,
        ],
    )
    def gather_kernel(table_hbm, idx_hbm, out_hbm, idx_v, gbuf, fbuf,
                      g0, g1, s0, s1):
        gs = (g0, g1)
        ss = (s0, s1)
        wid = lax.axis_index("s") * 2 + lax.axis_index("c")
        pltpu.sync_copy(idx_hbm.at[wid], idx_v)
        base = wid * n_per_w
        mask_hi = jnp.int32(-65536)

        def start_gather(jf, b):
            pltpu.async_copy(table_hbm.at[idx_v.at[jf]], gbuf.at[b], gs[b])

        def wait_gather(j, b):
            pltpu.make_async_copy(table_hbm.at[idx_v.at[j]], gbuf.at[b],
                                  gs[b]).wait()

        def start_scatter(j, b):
            pltpu.async_copy(fbuf.at[b],
                             out_hbm.at[pl.ds(base + j * CHUNK, CHUNK)], ss[b])

        def wait_scatter(j, b):
            pltpu.make_async_copy(fbuf.at[b],
                                  out_hbm.at[pl.ds(base + j * CHUNK, CHUNK)],
                                  ss[b]).wait()

        def expand(b):
            g = gbuf.at[b]
            f = fbuf.at[b]

            @plsc.parallel_loop(0, CHUNK, 1, unroll=2)
            def row(r):
                for sl in range(D2 // L):
                    w = g[r, pl.ds(sl * L, L)]
                    f[r, pl.ds(sl * L, L)] = w << 16
                    f[r, pl.ds(D2 + sl * L, L)] = w & mask_hi

        def step(j, b, first, last):
            wait_gather(j, b)
            if not first:
                wait_scatter(j - 2, b)
            expand(b)
            start_scatter(j, b)
            if not last:
                start_gather(j + 2, b)

        start_gather(0, 0)
        start_gather(1, 1)
        step(0, 0, True, False)
        step(1, 1, True, False)

        def body(kk, carry):
            step(kk * 2, 0, False, False)
            step(kk * 2 + 1, 1, False, False)
            return carry

        lax.fori_loop(1, n_chunks // 2 - 1, body, 0)

        step(n_chunks - 2, 0, False, True)
        step(n_chunks - 1, 1, False, True)
        wait_scatter(n_chunks - 2, 0)
        wait_scatter(n_chunks - 1, 1)

    return gather_kernel


def kernel(positions, positional_embeddings):
    b = positions.size
    n_per_w = b // NW
    n_chunks = n_per_w // CHUNK
    packed = _normalize_pack_table(positional_embeddings)
    idx = positions.reshape(NW, n_chunks, CHUNK).astype(jnp.int32)
    out = _make_sc_gather(b)(packed, idx)
    out = jax.lax.bitcast_convert_type(out, jnp.float32)
    return out.reshape(positions.shape + (D,))


# speedup vs baseline: 2.1107x; 1.5277x over previous
import functools

import jax
import jax.numpy as jnp
from jax import lax
from jax.experimental import pallas as pl
from jax.experimental.pallas import tpu as pltpu
from jax.experimental.pallas import tpu_sc as plsc

D = 1024
NW = 32
CHUNK = 32
NBUF = 3
LOOKAHEAD = 2


def _normalize_rows_body(t_ref, o_ref):
    x = t_ref[...]
    s = jnp.sum(x * x, axis=1, keepdims=True)
    norm = jnp.sqrt(s)
    o_ref[...] = x * (1.0 / jnp.maximum(norm, 1e-12))


def _normalize_table(table):
    rows, d = table.shape
    blk = 2048
    return pl.pallas_call(
        _normalize_rows_body,
        grid=(rows // blk,),
        in_specs=[pl.BlockSpec((blk, d), lambda i: (i, 0))],
        out_specs=pl.BlockSpec((blk, d), lambda i: (i, 0)),
        out_shape=jax.ShapeDtypeStruct((rows, d), table.dtype),
    )(table)


def _make_sc_gather(n_rows_total):
    n_per_w = n_rows_total // NW
    n_chunks = n_per_w // CHUNK
    assert n_chunks >= 3 * NBUF
    assert LOOKAHEAD <= NBUF
    n_full = (n_chunks // NBUF) * NBUF
    mesh = plsc.VectorSubcoreMesh(core_axis_name="c", subcore_axis_name="s")

    @functools.partial(
        pl.kernel,
        mesh=mesh,
        out_type=jax.ShapeDtypeStruct((n_rows_total, D), jnp.float32),
        scratch_types=[
            pltpu.VMEM((n_chunks, CHUNK), jnp.int32),
            pltpu.VMEM((NBUF, CHUNK, D), jnp.float32),
        ] + [pltpu.SemaphoreType.DMA] * (2 * NBUF),
    )
    def gather_kernel(table_hbm, idx_hbm, out_hbm, idx_v, buf, *sems):
        gs = sems[:NBUF]
        ss = sems[NBUF:]
        wid = lax.axis_index("s") * 2 + lax.axis_index("c")
        pltpu.sync_copy(idx_hbm.at[wid], idx_v)
        base = wid * n_per_w

        def start_gather(jf, b):
            pltpu.async_copy(table_hbm.at[idx_v.at[jf]], buf.at[b], gs[b])

        def wait_gather(j, b):
            pltpu.make_async_copy(table_hbm.at[idx_v.at[j]], buf.at[b],
                                  gs[b]).wait()

        def start_scatter(j, b):
            pltpu.async_copy(buf.at[b],
                             out_hbm.at[pl.ds(base + j * CHUNK, CHUNK)], ss[b])

        def wait_scatter(j, b):
            pltpu.make_async_copy(buf.at[b],
                                  out_hbm.at[pl.ds(base + j * CHUNK, CHUNK)],
                                  ss[b]).wait()

        def step(j, b):
            wait_gather(j, b)
            start_scatter(j, b)
            jf = j + LOOKAHEAD
            do_feed = (jf < n_chunks) if isinstance(j, int) else True
            if do_feed:
                bf = (b + LOOKAHEAD) % NBUF
                js = jf - NBUF
                do_drain = (js >= 0) if isinstance(j, int) else True
                if do_drain:
                    wait_scatter(js, bf)
                start_gather(jf, bf)

        for j in range(LOOKAHEAD):
            start_gather(j, j % NBUF)
        for j in range(NBUF):
            step(j, j)

        def body(kk, carry):
            for b in range(NBUF):
                step(kk * NBUF + b, b)
            return carry

        lax.fori_loop(1, n_full // NBUF - 1, body, 0)

        for j in range(n_full - NBUF, n_chunks):
            step(j, j % NBUF)
        for j in range(n_chunks - NBUF, n_chunks):
            wait_scatter(j, j % NBUF)

    return gather_kernel


def kernel(positions, positional_embeddings):
    b = positions.size
    n_per_w = b // NW
    n_chunks = n_per_w // CHUNK
    norm_table = _normalize_table(positional_embeddings)
    idx = positions.reshape(NW, n_chunks, CHUNK).astype(jnp.int32)
    out = _make_sc_gather(b)(norm_table, idx)
    return out.reshape(positions.shape + (D,))
